# baseline (speedup 1.0000x reference)
import functools

import jax
import jax.numpy as jnp
from jax import lax
from jax.experimental import pallas as pl
from jax.experimental.pallas import tpu as pltpu
from jax.experimental.pallas import tpu_sc as plsc

BATCH = 16384
EMB = 64
HID = 10

_info = plsc.get_sparse_core_info()
_NC, _NS = _info.num_cores, _info.num_subcores
NW = _NC * _NS
BPW = BATCH // NW
CHUNK = 128


def _sc_gather_body(user_hbm, movie_hbm, ut_hbm, mt_hbm, ue_hbm, me_hbm,
                    idx_u, idx_m, rows_u, rows_m, sem):
    wid = lax.axis_index("s") * _NC + lax.axis_index("c")
    base = wid * BPW
    pltpu.sync_copy(user_hbm.at[pl.ds(base, BPW)], idx_u)
    pltpu.sync_copy(movie_hbm.at[pl.ds(base, BPW)], idx_m)
    handles = []
    for c in range(BPW // CHUNK):
        sl = pl.ds(c * CHUNK, CHUNK)
        handles.append(pltpu.async_copy(ut_hbm.at[idx_u.at[sl]], rows_u.at[sl], sem))
        handles.append(pltpu.async_copy(mt_hbm.at[idx_m.at[sl]], rows_m.at[sl], sem))
    for h in handles:
        h.wait()
    pltpu.sync_copy(rows_u, ue_hbm.at[pl.ds(base, BPW)])
    pltpu.sync_copy(rows_m, me_hbm.at[pl.ds(base, BPW)])


_sc_gather = functools.partial(
    pl.kernel,
    out_type=[
        jax.ShapeDtypeStruct((BATCH, EMB), jnp.float32),
        jax.ShapeDtypeStruct((BATCH, EMB), jnp.float32),
    ],
    mesh=plsc.VectorSubcoreMesh(core_axis_name="c", subcore_axis_name="s"),
    scratch_types=[
        pltpu.VMEM((BPW,), jnp.int32),
        pltpu.VMEM((BPW,), jnp.int32),
        pltpu.VMEM((BPW, EMB), jnp.float32),
        pltpu.VMEM((BPW, EMB), jnp.float32),
        pltpu.SemaphoreType.DMA,
    ],
    compiler_params=pltpu.CompilerParams(use_tc_tiling_on_sc=False),
)(_sc_gather_body)


def _mlp_body(ue_ref, me_ref, w1u_ref, w1m_ref, b1_ref, w2_ref, b2_ref, out_ref):
    ue = jnp.maximum(ue_ref[...], 0.0)
    me = jnp.maximum(me_ref[...], 0.0)
    h = (
        jnp.dot(ue, w1u_ref[...], preferred_element_type=jnp.float32)
        + jnp.dot(me, w1m_ref[...], preferred_element_type=jnp.float32)
        + b1_ref[...]
    )
    h = jnp.maximum(h, 0.0)
    out_ref[...] = jnp.dot(h, w2_ref[...], preferred_element_type=jnp.float32) + b2_ref[...]


def _mlp(ue, me, w1u, w1m, b1, w2, b2):
    blk = 2048
    grid = (BATCH // blk,)
    return pl.pallas_call(
        _mlp_body,
        grid=grid,
        in_specs=[
            pl.BlockSpec((blk, EMB), lambda i: (i, 0)),
            pl.BlockSpec((blk, EMB), lambda i: (i, 0)),
            pl.BlockSpec((EMB, HID), lambda i: (0, 0)),
            pl.BlockSpec((EMB, HID), lambda i: (0, 0)),
            pl.BlockSpec((1, HID), lambda i: (0, 0)),
            pl.BlockSpec((HID, 1), lambda i: (0, 0)),
            pl.BlockSpec((1, 1), lambda i: (0, 0)),
        ],
        out_specs=pl.BlockSpec((blk, 1), lambda i: (i, 0)),
        out_shape=jax.ShapeDtypeStruct((BATCH, 1), jnp.float32),
    )(ue, me, w1u, w1m, b1, w2, b2)


def kernel(user, movie, user_table, movie_table, W1, b1, W2, b2):
    user = user.astype(jnp.int32)
    movie = movie.astype(jnp.int32)
    ue, me = _sc_gather(user, movie, user_table, movie_table)
    w1u = W1[:, :EMB].T
    w1m = W1[:, EMB:].T
    return _mlp(ue, me, w1u, w1m, b1.reshape(1, HID), W2.T, b2.reshape(1, 1))
